# baseline (speedup 1.0000x reference)
import functools

import jax
import jax.numpy as jnp
from jax import lax
from jax.experimental import pallas as pl
from jax.experimental.pallas import tpu as pltpu
from jax.experimental.pallas import tpu_sc as plsc

_N = 10000
_E = 320000
_D = 128
_DOUT = 40
_EPS = 1e-5

_NPAD = 10240
_NH = 4096
_NPASS = 3
_NTILES = 16
_C = 128
_EPAD = 327680
_NCORES = 2
_NW = _NCORES * _NTILES
_NCHUNK = _EPAD // (_NW * _C)
_RPT = _NH // _NTILES
_RB = 1024
_SHIFT = 14

_MESH = dict(core_axis_name="c", subcore_axis_name="s", num_cores=2)


def _make_scatter_kernel():
    @functools.partial(
        pl.kernel,
        mesh=plsc.VectorSubcoreMesh(**_MESH),
        out_type=jax.ShapeDtypeStruct((_NCORES, _NPASS * _NH, _D), jnp.float32),
        scratch_types=[
            pltpu.VMEM((_NCHUNK, _C), jnp.int32),
            pltpu.VMEM((_NCHUNK, _C), jnp.int32),
            pltpu.VMEM((_NCHUNK, _C), jnp.int32),
            pltpu.VMEM((_C, _D), jnp.float32),
            pltpu.VMEM((_C, _D), jnp.float32),
            pltpu.VMEM_SHARED((_NH, _D), jnp.float32),
            pltpu.SemaphoreType.DMA,
            pltpu.SemaphoreType.DMA,
        ],
    )
    def scatter_kernel(u_hbm, pk_hbm, out_hbm,
                       pk_v, src_v, dst_v, buf0, buf1, acc, sem0, sem1):
        cid = lax.axis_index("c")
        sid = lax.axis_index("s")
        wid = sid * _NCORES + cid
        base = sid * _RPT

        pltpu.sync_copy(pk_hbm.at[wid], pk_v)

        for r in range(_NPASS):
            def _unpack(i, carry):
                for j in range(_C // 16):
                    pk = pk_v[i, pl.ds(j * 16, 16)]
                    src = pk >> _SHIFT
                    dl = (pk & ((1 << _SHIFT) - 1)) - r * _NH
                    valid = (dl >= 0) & (dl < _NH)
                    src_v[i, pl.ds(j * 16, 16)] = jnp.where(
                        valid, src, _NPAD - 1)
                    dst_v[i, pl.ds(j * 16, 16)] = jnp.where(valid, dl, 0)
                return carry
            lax.fori_loop(0, _NCHUNK, _unpack, 0)

            def _zero(i, carry):
                for j in range(_D // 16):
                    buf0[i, pl.ds(j * 16, 16)] = jnp.zeros((16,), jnp.float32)
                return carry
            lax.fori_loop(0, _C, _zero, 0)
            for k in range(_RPT // _C):
                pltpu.sync_copy(buf0, acc.at[pl.ds(base + k * _C, _C)])
            plsc.subcore_barrier()

            pltpu.async_copy(u_hbm.at[src_v.at[0]], buf0, sem0)

            def _body(t, carry):
                j0 = 2 * t
                j1 = j0 + 1
                pltpu.async_copy(u_hbm.at[src_v.at[j1]], buf1, sem1)
                pltpu.make_async_copy(u_hbm.at[src_v.at[j0]], buf0,
                                      sem0).wait()
                pltpu.sync_copy(buf0, acc.at[dst_v.at[j0]], add=True)
                j2 = jnp.where(j1 + 1 >= _NCHUNK, 0, j1 + 1)
                pltpu.async_copy(u_hbm.at[src_v.at[j2]], buf0, sem0)
                pltpu.make_async_copy(u_hbm.at[src_v.at[j1]], buf1,
                                      sem1).wait()
                pltpu.sync_copy(buf1, acc.at[dst_v.at[j1]], add=True)
                return carry
            lax.fori_loop(0, _NCHUNK // 2, _body, 0)
            pltpu.make_async_copy(u_hbm.at[src_v.at[0]], buf0, sem0).wait()
            plsc.subcore_barrier()

            pltpu.sync_copy(acc.at[pl.ds(base, _RPT)],
                            out_hbm.at[cid, pl.ds(r * _NH + base, _RPT)])

    return scatter_kernel


_scatter = _make_scatter_kernel()


def _pack_body(src_ref, dst_ref, pk_ref):
    pk_ref[...] = (src_ref[...] << _SHIFT) | dst_ref[...]


def _pack(src, dst):
    rows = _EPAD // _C
    return pl.pallas_call(
        _pack_body,
        grid=(1,),
        in_specs=[
            pl.BlockSpec((rows, _C), lambda i: (0, 0)),
            pl.BlockSpec((rows, _C), lambda i: (0, 0)),
        ],
        out_specs=pl.BlockSpec((rows, _C), lambda i: (0, 0)),
        out_shape=jax.ShapeDtypeStruct((rows, _C), jnp.int32),
    )(src.reshape(rows, _C), dst.reshape(rows, _C))


def _mm_body(h_ref, w_ref, dinv_ref, u_ref):
    i = pl.program_id(0)
    row = i * _RB + lax.broadcasted_iota(jnp.int32, (_RB, 1), 0)
    u = dinv_ref[...] * jnp.dot(h_ref[...], w_ref[...],
                                preferred_element_type=jnp.float32)
    u_ref[...] = jnp.where(row < _N, u, 0.0)


def _mm(h, w, dinv):
    return pl.pallas_call(
        _mm_body,
        grid=(_NPAD // _RB,),
        in_specs=[
            pl.BlockSpec((_RB, _D), lambda i: (i, 0)),
            pl.BlockSpec((_D, _D), lambda i: (0, 0)),
            pl.BlockSpec((_RB, 1), lambda i: (i, 0)),
        ],
        out_specs=pl.BlockSpec((_RB, _D), lambda i: (i, 0)),
        out_shape=jax.ShapeDtypeStruct((_NPAD, _D), jnp.float32),
    )(h, w, dinv)


def _epi_body(p_ref, u_ref, dinv_ref, x_ref, b_ref, g_ref, be_ref,
              rm_ref, rv_ref, fl_ref, m_ref, h_ref, dinv_out_ref):
    psum = p_ref[0] + p_ref[1]
    t = dinv_ref[...] * (psum + u_ref[...]) + b_ref[...]
    s = g_ref[...] * lax.rsqrt(rv_ref[...] + _EPS)
    v = (t - rm_ref[...]) * s + be_ref[...]
    act = jnp.where(fl_ref[0, 0] > 0.0, jnp.maximum(v, 0.0), v)
    m = m_ref[0, 0] > 0.0
    h_ref[...] = jnp.where(m, act, x_ref[...])
    dinv_out_ref[...] = jnp.where(
        m, dinv_ref[...], lax.rsqrt(jnp.maximum(v[:, 0:1], 1.0)))


def _epi(p, u, dinv, x, b, g, be, rm, rv, fl, m):
    vec = lambda: pl.BlockSpec((1, _D), lambda i: (0, 0))
    one = lambda: pl.BlockSpec((1, 1), lambda i: (0, 0))
    return pl.pallas_call(
        _epi_body,
        grid=(_NPAD // _RB,),
        in_specs=[
            pl.BlockSpec((_NCORES, _RB, _D), lambda i: (0, i, 0)),
            pl.BlockSpec((_RB, _D), lambda i: (i, 0)),
            pl.BlockSpec((_RB, 1), lambda i: (i, 0)),
            pl.BlockSpec((_RB, _D), lambda i: (i, 0)),
            vec(), vec(), vec(), vec(), vec(),
            one(), one(),
        ],
        out_specs=[
            pl.BlockSpec((_RB, _D), lambda i: (i, 0)),
            pl.BlockSpec((_RB, 1), lambda i: (i, 0)),
        ],
        out_shape=[
            jax.ShapeDtypeStruct((_NPAD, _D), jnp.float32),
            jax.ShapeDtypeStruct((_NPAD, 1), jnp.float32),
        ],
    )(p, u, dinv, x, b, g, be, rm, rv, fl, m)


def _final_body(h_ref, o_ref):
    t = h_ref[...]
    col = lax.broadcasted_iota(jnp.int32, (_RB, _D), 1)
    valid = col < _DOUT
    tm = jnp.where(valid, t, -jnp.inf)
    mx = jnp.max(tm, axis=1, keepdims=True)
    e = jnp.where(valid, jnp.exp(t - mx), 0.0)
    lse = jnp.log(jnp.sum(e, axis=1, keepdims=True))
    o_ref[...] = t - mx - lse


def _final(h):
    return pl.pallas_call(
        _final_body,
        grid=(_NPAD // _RB,),
        in_specs=[pl.BlockSpec((_RB, _D), lambda i: (i, 0))],
        out_specs=pl.BlockSpec((_RB, _D), lambda i: (i, 0)),
        out_shape=jax.ShapeDtypeStruct((_NPAD, _D), jnp.float32),
    )(h)


def kernel(x, edge_index, W1, b1, g1, be1, rm1, rv1,
           W2, b2, g2, be2, rm2, rv2, W3, b3):
    f32 = jnp.float32
    x_pad = jnp.zeros((_NPAD, _D), f32).at[:_N].set(x)
    fill = jnp.full((_EPAD - _E,), _NPAD - 1, dtype=jnp.int32)
    srcp = jnp.concatenate([edge_index[0].astype(jnp.int32), fill])
    dstp = jnp.concatenate([edge_index[1].astype(jnp.int32), fill])

    pk = _pack(srcp, dstp)
    pk = pk.reshape(_NW, _NCHUNK, _C)

    W3p = jnp.zeros((_D, _D), f32).at[:, :_DOUT].set(W3)
    b3p = jnp.zeros((_D,), f32).at[:_DOUT].set(b3)
    ones = jnp.ones((_D,), f32)
    zeros = jnp.zeros((_D,), f32)
    rvid = jnp.full((_D,), 1.0 - _EPS, f32)
    eye = jnp.eye(_D, dtype=f32)
    Ws = jnp.stack([eye, W1, W2, W3p])
    bs = jnp.stack([zeros, b1, b2, b3p]).reshape(4, 1, _D)
    gs = jnp.stack([ones, g1, g2, ones]).reshape(4, 1, _D)
    bes = jnp.stack([zeros, be1, be2, zeros]).reshape(4, 1, _D)
    rms = jnp.stack([zeros, rm1, rm2, zeros]).reshape(4, 1, _D)
    rvs = jnp.stack([rvid, rv1, rv2, rvid]).reshape(4, 1, _D)
    fls = jnp.array([0.0, 1.0, 1.0, 0.0], f32).reshape(4, 1, 1)
    ms = jnp.array([0.0, 1.0, 1.0, 1.0], f32).reshape(4, 1, 1)

    def body(carry, xs):
        h, dinv = carry
        w, b, g, be, rm, rv, fl, m = xs
        u = _mm(h, w, dinv)
        p = _scatter(u, pk)
        h2, dinv2 = _epi(p, u, dinv, x_pad, b, g, be, rm, rv, fl, m)
        return (h2, dinv2), None

    carry0 = (jnp.ones((_NPAD, _D), f32), jnp.ones((_NPAD, 1), f32))
    (h3, _), _ = lax.scan(body, carry0,
                          (Ws, bs, gs, bes, rms, rvs, fls, ms))
    out = _final(h3)
    return out[:_N, :_DOUT]


# speedup vs baseline: 39.8897x; 39.8897x over previous
import functools

import jax
import jax.numpy as jnp
from jax import lax
from jax.experimental import pallas as pl
from jax.experimental.pallas import tpu as pltpu
from jax.experimental.pallas import tpu_sc as plsc

_N = 10000
_E = 320000
_D = 128
_DOUT = 40
_EPS = 1e-5

_NPAD = 10240
_NH = 4096
_NPASS = 3
_NTILES = 16
_C = 128
_EPAD = 327680
_NCORES = 2
_NW = _NCORES * _NTILES
_NCHUNK = _EPAD // (_NW * _C)
_RPT = _NH // _NTILES
_RB = 1024
_SHIFT = 14

_MESH = dict(core_axis_name="c", subcore_axis_name="s", num_cores=2)


def _make_scatter_kernel():
    @functools.partial(
        pl.kernel,
        mesh=plsc.VectorSubcoreMesh(**_MESH),
        out_type=jax.ShapeDtypeStruct((_NCORES, _NPASS * _NH, _D), jnp.float32),
        scratch_types=[
            pltpu.VMEM((_NCHUNK, _C), jnp.int32),
            pltpu.VMEM((_NCHUNK, _C), jnp.int32),
            pltpu.VMEM((_NCHUNK, _C), jnp.int32),
            pltpu.VMEM((_C, _D), jnp.float32),
            pltpu.VMEM((_C, _D), jnp.float32),
            pltpu.VMEM_SHARED((_NH, _D), jnp.float32),
            pltpu.SemaphoreType.DMA,
            pltpu.SemaphoreType.DMA,
        ],
    )
    def scatter_kernel(u_hbm, pk_hbm, out_hbm,
                       pk_v, src_v, dst_v, buf0, buf1, acc, sem0, sem1):
        cid = lax.axis_index("c")
        sid = lax.axis_index("s")
        wid = sid * _NCORES + cid
        base = sid * _RPT

        pltpu.sync_copy(pk_hbm.at[wid], pk_v)

        for r in range(_NPASS):
            def _unpack(i, carry):
                for j in range(_C // 16):
                    pk = pk_v[i, pl.ds(j * 16, 16)]
                    src = pk >> _SHIFT
                    dl = (pk & ((1 << _SHIFT) - 1)) - r * _NH
                    valid = (dl >= 0) & (dl < _NH)
                    src_v[i, pl.ds(j * 16, 16)] = jnp.where(valid, src, -1)
                    dst_v[i, pl.ds(j * 16, 16)] = jnp.where(valid, dl, -1)
                return carry
            lax.fori_loop(0, _NCHUNK, _unpack, 0)

            def _zero(i, carry):
                for j in range(_D // 16):
                    buf0[i, pl.ds(j * 16, 16)] = jnp.zeros((16,), jnp.float32)
                return carry
            lax.fori_loop(0, _C, _zero, 0)
            for k in range(_RPT // _C):
                pltpu.sync_copy(buf0, acc.at[pl.ds(base + k * _C, _C)])
            plsc.subcore_barrier()

            gidx = lambda j: plsc.Indices(src_v.at[j], ignored_value=-1)
            sidx = lambda j: plsc.Indices(dst_v.at[j], ignored_value=-1)
            pltpu.async_copy(u_hbm.at[gidx(0)], buf0, sem0)

            def _body(t, carry):
                j0 = 2 * t
                j1 = j0 + 1
                pltpu.async_copy(u_hbm.at[gidx(j1)], buf1, sem1)
                pltpu.make_async_copy(u_hbm.at[gidx(j0)], buf0, sem0).wait()
                pltpu.sync_copy(buf0, acc.at[sidx(j0)], add=True)
                j2 = jnp.where(j1 + 1 >= _NCHUNK, 0, j1 + 1)
                pltpu.async_copy(u_hbm.at[gidx(j2)], buf0, sem0)
                pltpu.make_async_copy(u_hbm.at[gidx(j1)], buf1, sem1).wait()
                pltpu.sync_copy(buf1, acc.at[sidx(j1)], add=True)
                return carry
            lax.fori_loop(0, _NCHUNK // 2, _body, 0)
            pltpu.make_async_copy(u_hbm.at[gidx(0)], buf0, sem0).wait()
            plsc.subcore_barrier()

            pltpu.sync_copy(acc.at[pl.ds(base, _RPT)],
                            out_hbm.at[cid, pl.ds(r * _NH + base, _RPT)])

    return scatter_kernel


_scatter = _make_scatter_kernel()


def _pack_body(src_ref, dst_ref, pk_ref):
    pk_ref[...] = (src_ref[...] << _SHIFT) | dst_ref[...]


def _pack(src, dst):
    rows = _EPAD // _C
    return pl.pallas_call(
        _pack_body,
        grid=(1,),
        in_specs=[
            pl.BlockSpec((rows, _C), lambda i: (0, 0)),
            pl.BlockSpec((rows, _C), lambda i: (0, 0)),
        ],
        out_specs=pl.BlockSpec((rows, _C), lambda i: (0, 0)),
        out_shape=jax.ShapeDtypeStruct((rows, _C), jnp.int32),
    )(src.reshape(rows, _C), dst.reshape(rows, _C))


def _mm_body(h_ref, w_ref, dinv_ref, u_ref):
    i = pl.program_id(0)
    row = i * _RB + lax.broadcasted_iota(jnp.int32, (_RB, 1), 0)
    u = dinv_ref[...] * jnp.dot(h_ref[...], w_ref[...],
                                preferred_element_type=jnp.float32)
    u_ref[...] = jnp.where(row < _N, u, 0.0)


def _mm(h, w, dinv):
    return pl.pallas_call(
        _mm_body,
        grid=(_NPAD // _RB,),
        in_specs=[
            pl.BlockSpec((_RB, _D), lambda i: (i, 0)),
            pl.BlockSpec((_D, _D), lambda i: (0, 0)),
            pl.BlockSpec((_RB, 1), lambda i: (i, 0)),
        ],
        out_specs=pl.BlockSpec((_RB, _D), lambda i: (i, 0)),
        out_shape=jax.ShapeDtypeStruct((_NPAD, _D), jnp.float32),
    )(h, w, dinv)


def _epi_body(p_ref, u_ref, dinv_ref, x_ref, b_ref, g_ref, be_ref,
              rm_ref, rv_ref, fl_ref, m_ref, h_ref, dinv_out_ref):
    psum = p_ref[0] + p_ref[1]
    t = dinv_ref[...] * (psum + u_ref[...]) + b_ref[...]
    s = g_ref[...] * lax.rsqrt(rv_ref[...] + _EPS)
    v = (t - rm_ref[...]) * s + be_ref[...]
    act = jnp.where(fl_ref[0, 0] > 0.0, jnp.maximum(v, 0.0), v)
    m = m_ref[0, 0] > 0.0
    h_ref[...] = jnp.where(m, act, x_ref[...])
    dinv_out_ref[...] = jnp.where(
        m, dinv_ref[...], lax.rsqrt(jnp.maximum(v[:, 0:1], 1.0)))


def _epi(p, u, dinv, x, b, g, be, rm, rv, fl, m):
    vec = lambda: pl.BlockSpec((1, _D), lambda i: (0, 0))
    one = lambda: pl.BlockSpec((1, 1), lambda i: (0, 0))
    return pl.pallas_call(
        _epi_body,
        grid=(_NPAD // _RB,),
        in_specs=[
            pl.BlockSpec((_NCORES, _RB, _D), lambda i: (0, i, 0)),
            pl.BlockSpec((_RB, _D), lambda i: (i, 0)),
            pl.BlockSpec((_RB, 1), lambda i: (i, 0)),
            pl.BlockSpec((_RB, _D), lambda i: (i, 0)),
            vec(), vec(), vec(), vec(), vec(),
            one(), one(),
        ],
        out_specs=[
            pl.BlockSpec((_RB, _D), lambda i: (i, 0)),
            pl.BlockSpec((_RB, 1), lambda i: (i, 0)),
        ],
        out_shape=[
            jax.ShapeDtypeStruct((_NPAD, _D), jnp.float32),
            jax.ShapeDtypeStruct((_NPAD, 1), jnp.float32),
        ],
    )(p, u, dinv, x, b, g, be, rm, rv, fl, m)


def _final_body(h_ref, o_ref):
    t = h_ref[...]
    col = lax.broadcasted_iota(jnp.int32, (_RB, _D), 1)
    valid = col < _DOUT
    tm = jnp.where(valid, t, -jnp.inf)
    mx = jnp.max(tm, axis=1, keepdims=True)
    e = jnp.where(valid, jnp.exp(t - mx), 0.0)
    lse = jnp.log(jnp.sum(e, axis=1, keepdims=True))
    o_ref[...] = t - mx - lse


def _final(h):
    return pl.pallas_call(
        _final_body,
        grid=(_NPAD // _RB,),
        in_specs=[pl.BlockSpec((_RB, _D), lambda i: (i, 0))],
        out_specs=pl.BlockSpec((_RB, _D), lambda i: (i, 0)),
        out_shape=jax.ShapeDtypeStruct((_NPAD, _D), jnp.float32),
    )(h)


def kernel(x, edge_index, W1, b1, g1, be1, rm1, rv1,
           W2, b2, g2, be2, rm2, rv2, W3, b3):
    f32 = jnp.float32
    x_pad = jnp.zeros((_NPAD, _D), f32).at[:_N].set(x)
    fill = jnp.full((_EPAD - _E,), _NPAD - 1, dtype=jnp.int32)
    srcp = jnp.concatenate([edge_index[0].astype(jnp.int32), fill])
    dstp = jnp.concatenate([edge_index[1].astype(jnp.int32), fill])

    pk = _pack(srcp, dstp)
    pk = pk.reshape(_NW, _NCHUNK, _C)

    W3p = jnp.zeros((_D, _D), f32).at[:, :_DOUT].set(W3)
    b3p = jnp.zeros((_D,), f32).at[:_DOUT].set(b3)
    ones = jnp.ones((_D,), f32)
    zeros = jnp.zeros((_D,), f32)
    rvid = jnp.full((_D,), 1.0 - _EPS, f32)
    eye = jnp.eye(_D, dtype=f32)
    Ws = jnp.stack([eye, W1, W2, W3p])
    bs = jnp.stack([zeros, b1, b2, b3p]).reshape(4, 1, _D)
    gs = jnp.stack([ones, g1, g2, ones]).reshape(4, 1, _D)
    bes = jnp.stack([zeros, be1, be2, zeros]).reshape(4, 1, _D)
    rms = jnp.stack([zeros, rm1, rm2, zeros]).reshape(4, 1, _D)
    rvs = jnp.stack([rvid, rv1, rv2, rvid]).reshape(4, 1, _D)
    fls = jnp.array([0.0, 1.0, 1.0, 0.0], f32).reshape(4, 1, 1)
    ms = jnp.array([0.0, 1.0, 1.0, 1.0], f32).reshape(4, 1, 1)

    def body(carry, xs):
        h, dinv = carry
        w, b, g, be, rm, rv, fl, m = xs
        u = _mm(h, w, dinv)
        p = _scatter(u, pk)
        h2, dinv2 = _epi(p, u, dinv, x_pad, b, g, be, rm, rv, fl, m)
        return (h2, dinv2), None

    carry0 = (jnp.ones((_NPAD, _D), f32), jnp.ones((_NPAD, 1), f32))
    (h3, _), _ = lax.scan(body, carry0,
                          (Ws, bs, gs, bes, rms, rvs, fls, ms))
    out = _final(h3)
    return out[:_N, :_DOUT]


# speedup vs baseline: 42.6532x; 1.0693x over previous
import functools

import jax
import jax.numpy as jnp
from jax import lax
from jax.experimental import pallas as pl
from jax.experimental.pallas import tpu as pltpu
from jax.experimental.pallas import tpu_sc as plsc

_N = 10000
_E = 320000
_D = 128
_DOUT = 40
_EPS = 1e-5

_NPAD = 10240
_NH = 4096
_NPASS = 3
_NTILES = 16
_C = 128
_EPAD = 327680
_NCORES = 2
_NW = _NCORES * _NTILES
_NCHUNK = _EPAD // (_NW * _C)
_RPT = _NH // _NTILES
_RB = 1024
_SHIFT = 14

_MESH = dict(core_axis_name="c", subcore_axis_name="s", num_cores=2)


def _make_scatter_kernel():
    @functools.partial(
        pl.kernel,
        mesh=plsc.VectorSubcoreMesh(**_MESH),
        out_type=jax.ShapeDtypeStruct((_NCORES, _NPASS * _NH, _D), jnp.float32),
        scratch_types=[
            pltpu.VMEM((_NCHUNK, _C), jnp.int32),
            pltpu.VMEM((_NCHUNK, _C), jnp.int32),
            pltpu.VMEM((_NCHUNK, _C), jnp.int32),
            pltpu.VMEM((_C, _D), jnp.float32),
            pltpu.VMEM((_C, _D), jnp.float32),
            pltpu.VMEM((_C, _D), jnp.float32),
            pltpu.VMEM((_C, _D), jnp.float32),
            pltpu.VMEM_SHARED((_NH, _D), jnp.float32),
            pltpu.SemaphoreType.DMA,
            pltpu.SemaphoreType.DMA,
            pltpu.SemaphoreType.DMA,
            pltpu.SemaphoreType.DMA,
            pltpu.SemaphoreType.DMA,
            pltpu.SemaphoreType.DMA,
            pltpu.SemaphoreType.DMA,
            pltpu.SemaphoreType.DMA,
        ],
    )
    def scatter_kernel(u_hbm, pk_hbm, out_hbm,
                       pk_v, src_v, dst_v, buf0, buf1, buf2, buf3, acc,
                       sg0, sg1, sg2, sg3, ss0, ss1, ss2, ss3):
        cid = lax.axis_index("c")
        sid = lax.axis_index("s")
        wid = sid * _NCORES + cid
        base = sid * _RPT

        pltpu.sync_copy(pk_hbm.at[wid], pk_v)

        for r in range(_NPASS):
            def _unpack(i, carry):
                for j in range(_C // 16):
                    pk = pk_v[i, pl.ds(j * 16, 16)]
                    src = pk >> _SHIFT
                    dl = (pk & ((1 << _SHIFT) - 1)) - r * _NH
                    valid = (dl >= 0) & (dl < _NH)
                    src_v[i, pl.ds(j * 16, 16)] = jnp.where(valid, src, -1)
                    dst_v[i, pl.ds(j * 16, 16)] = jnp.where(valid, dl, -1)
                return carry
            lax.fori_loop(0, _NCHUNK, _unpack, 0)

            def _zero(i, carry):
                for j in range(_D // 16):
                    buf0[i, pl.ds(j * 16, 16)] = jnp.zeros((16,), jnp.float32)
                return carry
            lax.fori_loop(0, _C, _zero, 0)
            for k in range(_RPT // _C):
                pltpu.sync_copy(buf0, acc.at[pl.ds(base + k * _C, _C)])
            plsc.subcore_barrier()

            gidx = lambda j: plsc.Indices(src_v.at[j], ignored_value=-1)
            sidx = lambda j: plsc.Indices(dst_v.at[j], ignored_value=-1)
            bufs = (buf0, buf1, buf2, buf3)
            sgs = (sg0, sg1, sg2, sg3)
            sss = (ss0, ss1, ss2, ss3)
            for b in range(4):
                pltpu.async_copy(u_hbm.at[gidx(b)], bufs[b], sgs[b])

            def _body(t, carry):
                jb = 4 * t
                for b in range(4):
                    j = jb + b
                    pltpu.make_async_copy(u_hbm.at[gidx(j)], bufs[b],
                                          sgs[b]).wait()
                    pltpu.async_copy(bufs[b], acc.at[sidx(j)], sss[b],
                                     add=True)
                for b in range(4):
                    j = jb + b
                    pltpu.make_async_copy(bufs[b], acc.at[sidx(j)],
                                          sss[b]).wait()
                    jn = jnp.where(j + 4 >= _NCHUNK, b, j + 4)
                    pltpu.async_copy(u_hbm.at[gidx(jn)], bufs[b], sgs[b])
                return carry
            lax.fori_loop(0, _NCHUNK // 4, _body, 0)
            for b in range(4):
                pltpu.make_async_copy(u_hbm.at[gidx(b)], bufs[b],
                                      sgs[b]).wait()
            plsc.subcore_barrier()

            pltpu.sync_copy(acc.at[pl.ds(base, _RPT)],
                            out_hbm.at[cid, pl.ds(r * _NH + base, _RPT)])

    return scatter_kernel


_scatter = _make_scatter_kernel()


def _pack_body(src_ref, dst_ref, pk_ref):
    pk_ref[...] = (src_ref[...] << _SHIFT) | dst_ref[...]


def _pack(src, dst):
    rows = _EPAD // _C
    return pl.pallas_call(
        _pack_body,
        grid=(1,),
        in_specs=[
            pl.BlockSpec((rows, _C), lambda i: (0, 0)),
            pl.BlockSpec((rows, _C), lambda i: (0, 0)),
        ],
        out_specs=pl.BlockSpec((rows, _C), lambda i: (0, 0)),
        out_shape=jax.ShapeDtypeStruct((rows, _C), jnp.int32),
    )(src.reshape(rows, _C), dst.reshape(rows, _C))


def _mm_body(h_ref, w_ref, dinv_ref, u_ref):
    i = pl.program_id(0)
    row = i * _RB + lax.broadcasted_iota(jnp.int32, (_RB, 1), 0)
    u = dinv_ref[...] * jnp.dot(h_ref[...], w_ref[...],
                                preferred_element_type=jnp.float32)
    u_ref[...] = jnp.where(row < _N, u, 0.0)


def _mm(h, w, dinv):
    return pl.pallas_call(
        _mm_body,
        grid=(_NPAD // _RB,),
        in_specs=[
            pl.BlockSpec((_RB, _D), lambda i: (i, 0)),
            pl.BlockSpec((_D, _D), lambda i: (0, 0)),
            pl.BlockSpec((_RB, 1), lambda i: (i, 0)),
        ],
        out_specs=pl.BlockSpec((_RB, _D), lambda i: (i, 0)),
        out_shape=jax.ShapeDtypeStruct((_NPAD, _D), jnp.float32),
    )(h, w, dinv)


def _epi_body(p_ref, u_ref, dinv_ref, x_ref, b_ref, g_ref, be_ref,
              rm_ref, rv_ref, fl_ref, m_ref, h_ref, dinv_out_ref):
    psum = p_ref[0] + p_ref[1]
    t = dinv_ref[...] * (psum + u_ref[...]) + b_ref[...]
    s = g_ref[...] * lax.rsqrt(rv_ref[...] + _EPS)
    v = (t - rm_ref[...]) * s + be_ref[...]
    act = jnp.where(fl_ref[0, 0] > 0.0, jnp.maximum(v, 0.0), v)
    m = m_ref[0, 0] > 0.0
    h_ref[...] = jnp.where(m, act, x_ref[...])
    dinv_out_ref[...] = jnp.where(
        m, dinv_ref[...], lax.rsqrt(jnp.maximum(v[:, 0:1], 1.0)))


def _epi(p, u, dinv, x, b, g, be, rm, rv, fl, m):
    vec = lambda: pl.BlockSpec((1, _D), lambda i: (0, 0))
    one = lambda: pl.BlockSpec((1, 1), lambda i: (0, 0))
    return pl.pallas_call(
        _epi_body,
        grid=(_NPAD // _RB,),
        in_specs=[
            pl.BlockSpec((_NCORES, _RB, _D), lambda i: (0, i, 0)),
            pl.BlockSpec((_RB, _D), lambda i: (i, 0)),
            pl.BlockSpec((_RB, 1), lambda i: (i, 0)),
            pl.BlockSpec((_RB, _D), lambda i: (i, 0)),
            vec(), vec(), vec(), vec(), vec(),
            one(), one(),
        ],
        out_specs=[
            pl.BlockSpec((_RB, _D), lambda i: (i, 0)),
            pl.BlockSpec((_RB, 1), lambda i: (i, 0)),
        ],
        out_shape=[
            jax.ShapeDtypeStruct((_NPAD, _D), jnp.float32),
            jax.ShapeDtypeStruct((_NPAD, 1), jnp.float32),
        ],
    )(p, u, dinv, x, b, g, be, rm, rv, fl, m)


def _final_body(h_ref, o_ref):
    t = h_ref[...]
    col = lax.broadcasted_iota(jnp.int32, (_RB, _D), 1)
    valid = col < _DOUT
    tm = jnp.where(valid, t, -jnp.inf)
    mx = jnp.max(tm, axis=1, keepdims=True)
    e = jnp.where(valid, jnp.exp(t - mx), 0.0)
    lse = jnp.log(jnp.sum(e, axis=1, keepdims=True))
    o_ref[...] = t - mx - lse


def _final(h):
    return pl.pallas_call(
        _final_body,
        grid=(_NPAD // _RB,),
        in_specs=[pl.BlockSpec((_RB, _D), lambda i: (i, 0))],
        out_specs=pl.BlockSpec((_RB, _D), lambda i: (i, 0)),
        out_shape=jax.ShapeDtypeStruct((_NPAD, _D), jnp.float32),
    )(h)


def kernel(x, edge_index, W1, b1, g1, be1, rm1, rv1,
           W2, b2, g2, be2, rm2, rv2, W3, b3):
    f32 = jnp.float32
    x_pad = jnp.zeros((_NPAD, _D), f32).at[:_N].set(x)
    fill = jnp.full((_EPAD - _E,), _NPAD - 1, dtype=jnp.int32)
    srcp = jnp.concatenate([edge_index[0].astype(jnp.int32), fill])
    dstp = jnp.concatenate([edge_index[1].astype(jnp.int32), fill])

    pk = _pack(srcp, dstp)
    pk = pk.reshape(_NW, _NCHUNK, _C)

    W3p = jnp.zeros((_D, _D), f32).at[:, :_DOUT].set(W3)
    b3p = jnp.zeros((_D,), f32).at[:_DOUT].set(b3)
    ones = jnp.ones((_D,), f32)
    zeros = jnp.zeros((_D,), f32)
    rvid = jnp.full((_D,), 1.0 - _EPS, f32)
    eye = jnp.eye(_D, dtype=f32)
    Ws = jnp.stack([eye, W1, W2, W3p])
    bs = jnp.stack([zeros, b1, b2, b3p]).reshape(4, 1, _D)
    gs = jnp.stack([ones, g1, g2, ones]).reshape(4, 1, _D)
    bes = jnp.stack([zeros, be1, be2, zeros]).reshape(4, 1, _D)
    rms = jnp.stack([zeros, rm1, rm2, zeros]).reshape(4, 1, _D)
    rvs = jnp.stack([rvid, rv1, rv2, rvid]).reshape(4, 1, _D)
    fls = jnp.array([0.0, 1.0, 1.0, 0.0], f32).reshape(4, 1, 1)
    ms = jnp.array([0.0, 1.0, 1.0, 1.0], f32).reshape(4, 1, 1)

    def body(carry, xs):
        h, dinv = carry
        w, b, g, be, rm, rv, fl, m = xs
        u = _mm(h, w, dinv)
        p = _scatter(u, pk)
        h2, dinv2 = _epi(p, u, dinv, x_pad, b, g, be, rm, rv, fl, m)
        return (h2, dinv2), None

    carry0 = (jnp.ones((_NPAD, _D), f32), jnp.ones((_NPAD, 1), f32))
    (h3, _), _ = lax.scan(body, carry0,
                          (Ws, bs, gs, bes, rms, rvs, fls, ms))
    out = _final(h3)
    return out[:_N, :_DOUT]
